# Initial kernel scaffold; baseline (speedup 1.0000x reference)
import functools

import jax
import jax.numpy as jnp
from jax import lax
from jax.experimental import pallas as pl
from jax.experimental.pallas import tpu as pltpu
from jax.experimental.pallas import tpu_sc as plsc

NN = 10000
D = 128
NG = 16
NCLS = 10
NP = 10240
NC, NS = 2, 16
NW = NC * NS
CHUNK = 128
ROWS_W = 80
EP = NW * ROWS_W * CHUNK
STRIPE = NP // NS

_mesh = plsc.VectorSubcoreMesh(core_axis_name="c", subcore_axis_name="s")


@functools.partial(
    pl.kernel,
    out_type=jax.ShapeDtypeStruct((NC, NP, 16), jnp.float32),
    mesh=_mesh,
    scratch_types=[
        pltpu.VMEM((ROWS_W, CHUNK), jnp.int32),
        pltpu.VMEM((CHUNK, 16), jnp.float32),
        pltpu.VMEM_SHARED((NP, 16), jnp.float32),
    ],
)
def _sc_degree(dst2d, ones, zeros16, degp, didx, ones_v, deg_sh):
    c = lax.axis_index("c")
    s = lax.axis_index("s")
    wid = c * NS + s
    pltpu.sync_copy(zeros16, deg_sh.at[pl.ds(s * STRIPE, STRIPE)])
    pltpu.sync_copy(ones, ones_v)
    pltpu.sync_copy(dst2d.at[pl.ds(wid * ROWS_W, ROWS_W)], didx)
    plsc.subcore_barrier()

    @pl.loop(0, ROWS_W)
    def _(k):
        pltpu.sync_copy(ones_v, deg_sh.at[didx.at[k]], add=True)

    plsc.subcore_barrier()
    pltpu.sync_copy(deg_sh.at[pl.ds(s * STRIPE, STRIPE)],
                    degp.at[c, pl.ds(s * STRIPE, STRIPE)])


@functools.partial(
    pl.kernel,
    out_type=jax.ShapeDtypeStruct((NC, NP, D), jnp.float32),
    mesh=_mesh,
    scratch_types=[
        pltpu.VMEM((ROWS_W, CHUNK), jnp.int32),
        pltpu.VMEM((ROWS_W, CHUNK), jnp.int32),
        pltpu.VMEM((CHUNK, D), jnp.float32),
        pltpu.VMEM((CHUNK, D), jnp.float32),
        pltpu.VMEM_SHARED((NP, D), jnp.float32),
        pltpu.SemaphoreType.DMA,
        pltpu.SemaphoreType.DMA,
    ],
)
def _sc_agg(src2d, dst2d, hs, zeros, part, sidx, didx, buf0, buf1, agg_sh,
            sem0, sem1):
    c = lax.axis_index("c")
    s = lax.axis_index("s")
    wid = c * NS + s
    pltpu.sync_copy(zeros, agg_sh.at[pl.ds(s * STRIPE, STRIPE)])
    pltpu.sync_copy(src2d.at[pl.ds(wid * ROWS_W, ROWS_W)], sidx)
    pltpu.sync_copy(dst2d.at[pl.ds(wid * ROWS_W, ROWS_W)], didx)
    plsc.subcore_barrier()

    pltpu.async_copy(hs.at[sidx.at[0]], buf0, sem0)

    @pl.loop(0, ROWS_W, step=2)
    def _(k):
        pltpu.make_async_copy(hs.at[sidx.at[k]], buf0, sem0).wait()
        pltpu.async_copy(hs.at[sidx.at[k + 1]], buf1, sem1)
        pltpu.sync_copy(buf0, agg_sh.at[didx.at[k]], add=True)
        pltpu.make_async_copy(hs.at[sidx.at[k + 1]], buf1, sem1).wait()

        @pl.when(k + 2 < ROWS_W)
        def _():
            pltpu.async_copy(hs.at[sidx.at[k + 2]], buf0, sem0)

        pltpu.sync_copy(buf1, agg_sh.at[didx.at[k + 1]], add=True)

    plsc.subcore_barrier()
    pltpu.sync_copy(agg_sh.at[pl.ds(s * STRIPE, STRIPE)],
                    part.at[c, pl.ds(s * STRIPE, STRIPE)])


def _tc_prep_body(x_ref, we_ref, be_ref, degp_ref, hs_ref, dinv_ref):
    x = x_ref[...]
    h = jnp.maximum(
        jnp.dot(x, we_ref[...], preferred_element_type=jnp.float32)
        + be_ref[...], 0.0)
    deg = degp_ref[0, :, 0] + degp_ref[1, :, 0] + 1.0
    dinv = lax.rsqrt(jnp.maximum(deg, 1.0))
    dinv_col = dinv[:, None]
    dinv_ref[...] = dinv_col
    hs_ref[0:NN, :] = h * dinv_col[0:NN, :]
    hs_ref[NN:NP, :] = jnp.zeros((NP - NN, D), jnp.float32)


def _tc_conv_body(part_ref, hs_ref, dinv_ref, w_ref, b_ref, g_ref, be_ref,
                  out_ref):
    dinv = dinv_ref[...]
    ssum = part_ref[0] + part_ref[1] + hs_ref[...]
    agg = ssum[0:NN] * dinv[0:NN]
    z = jnp.dot(agg, w_ref[...], preferred_element_type=jnp.float32) + b_ref[...]
    mu = jnp.mean(z, axis=0, keepdims=True)
    zc = z - mu
    var = jnp.mean(zc * zc, axis=0, keepdims=True)
    h = jnp.maximum(zc * lax.rsqrt(var + 1e-5) * g_ref[...] + be_ref[...], 0.0)
    out_ref[0:NN, :] = h * dinv[0:NN]
    out_ref[NN:NP, :] = jnp.zeros((NP - NN, D), jnp.float32)


def _tc_final_body(part_ref, hs_ref, dinv_ref, w3_ref, b3_ref, batch_ref,
                   wd1_ref, bd1_ref, wd2_ref, bd2_ref, out_ref):
    dinv = dinv_ref[...]
    ssum = part_ref[0] + part_ref[1] + hs_ref[...]
    agg = ssum[0:NN] * dinv[0:NN]
    z = jnp.dot(agg, w3_ref[...], preferred_element_type=jnp.float32) + b3_ref[...]
    b = batch_ref[...]
    gi = lax.broadcasted_iota(jnp.int32, (NG, NN), 0)
    oh = jnp.where(b == gi, 1.0, 0.0)
    ssel = jnp.dot(oh, z, preferred_element_type=jnp.float32)
    cnt = jnp.sum(oh, axis=1, keepdims=True)
    pooled = ssel / jnp.maximum(cnt, 1.0)
    t = jnp.maximum(
        jnp.dot(pooled, wd1_ref[...], preferred_element_type=jnp.float32)
        + bd1_ref[...], 0.0)
    out_ref[...] = (jnp.dot(t, wd2_ref[...], preferred_element_type=jnp.float32)
                    + bd2_ref[...])


def _f32(shape):
    return jax.ShapeDtypeStruct(shape, jnp.float32)


@jax.jit
def _impl(x, edge_index, batch, W_enc, b_enc, W1, b1, g1, be1, W2, b2, g2,
          be2, W3, b3, Wd1, bd1, Wd2, bd2):
    src = edge_index[0]
    dst = edge_index[1]
    e = src.shape[0]
    pad = EP - e
    src2 = jnp.concatenate(
        [src, jnp.zeros((pad,), jnp.int32)]).reshape(EP // CHUNK, CHUNK)
    dst2 = jnp.concatenate(
        [dst, jnp.full((pad,), NN, jnp.int32)]).reshape(EP // CHUNK, CHUNK)
    zeros = jnp.zeros((STRIPE, D), jnp.float32)
    zeros16 = jnp.zeros((STRIPE, 16), jnp.float32)
    ones16 = jnp.ones((CHUNK, 16), jnp.float32)

    degp = _sc_degree(dst2, ones16, zeros16)
    hs, dinv = pl.pallas_call(
        _tc_prep_body,
        out_shape=[_f32((NP, D)), _f32((NP, 1))],
    )(x, W_enc, b_enc.reshape(1, D), degp)

    for (W, bb, g, be) in ((W1, b1, g1, be1), (W2, b2, g2, be2)):
        part = _sc_agg(src2, dst2, hs, zeros)
        hs = pl.pallas_call(
            _tc_conv_body,
            out_shape=_f32((NP, D)),
        )(part, hs, dinv, W, bb.reshape(1, D), g.reshape(1, D),
          be.reshape(1, D))

    part = _sc_agg(src2, dst2, hs, zeros)
    out = pl.pallas_call(
        _tc_final_body,
        out_shape=_f32((NG, NCLS)),
    )(part, hs, dinv, W3, b3.reshape(1, D), batch.reshape(1, NN), Wd1,
      bd1.reshape(1, D), Wd2, bd2.reshape(1, NCLS))
    return out


def kernel(x, edge_index, batch, W_enc, b_enc, W1, b1, g1, be1, W2, b2, g2,
           be2, W3, b3, Wd1, bd1, Wd2, bd2):
    return _impl(x, edge_index, batch, W_enc, b_enc, W1, b1, g1, be1, W2, b2,
                 g2, be2, W3, b3, Wd1, bd1, Wd2, bd2)


# speedup vs baseline: 10.0433x; 10.0433x over previous
import functools

import jax
import jax.numpy as jnp
from jax import lax
from jax.experimental import pallas as pl
from jax.experimental.pallas import tpu as pltpu
from jax.experimental.pallas import tpu_sc as plsc

NN = 10000
D = 128
DH = D // 2
NG = 16
NCLS = 10
NP = 10240
NC, NS = 2, 16
NW = NC * NS
CHUNK = 128
EP = 327680
ROWS_S = EP // NS // CHUNK
ROWS_W = EP // NW // CHUNK
STRIPE = NP // NS

_mesh = plsc.VectorSubcoreMesh(core_axis_name="c", subcore_axis_name="s")


@functools.partial(
    pl.kernel,
    out_type=jax.ShapeDtypeStruct((NC, NP, 16), jnp.float32),
    mesh=_mesh,
    compiler_params=pltpu.CompilerParams(use_tc_tiling_on_sc=False),
    scratch_types=[
        pltpu.VMEM((ROWS_W, CHUNK), jnp.int32),
        pltpu.VMEM((CHUNK, 16), jnp.float32),
        pltpu.VMEM_SHARED((NP, 16), jnp.float32),
    ],
)
def _sc_degree(dst2d, ones, zeros16, degp, didx, ones_v, deg_sh):
    c = lax.axis_index("c")
    s = lax.axis_index("s")
    wid = c * NS + s
    pltpu.sync_copy(zeros16, deg_sh.at[pl.ds(s * STRIPE, STRIPE)])
    pltpu.sync_copy(ones, ones_v)
    pltpu.sync_copy(dst2d.at[pl.ds(wid * ROWS_W, ROWS_W)], didx)
    plsc.subcore_barrier()

    @pl.loop(0, ROWS_W)
    def _(k):
        pltpu.sync_copy(ones_v, deg_sh.at[didx.at[k]], add=True)

    plsc.subcore_barrier()
    pltpu.sync_copy(deg_sh.at[pl.ds(s * STRIPE, STRIPE)],
                    degp.at[c, pl.ds(s * STRIPE, STRIPE)])


@functools.partial(
    pl.kernel,
    out_type=jax.ShapeDtypeStruct((NC, NP, DH), jnp.float32),
    mesh=_mesh,
    compiler_params=pltpu.CompilerParams(use_tc_tiling_on_sc=False),
    scratch_types=[
        pltpu.VMEM((ROWS_S, CHUNK), jnp.int32),
        pltpu.VMEM((ROWS_S, CHUNK), jnp.int32),
        pltpu.VMEM((CHUNK, DH), jnp.float32),
        pltpu.VMEM((CHUNK, DH), jnp.float32),
        pltpu.VMEM_SHARED((NP, DH), jnp.float32),
        pltpu.SemaphoreType.DMA,
        pltpu.SemaphoreType.DMA,
    ],
)
def _sc_agg(src2d, dst2d, hs_a, hs_b, zeros, part, sidx, didx, buf0, buf1,
            agg_sh, sem0, sem1):
    c = lax.axis_index("c")
    s = lax.axis_index("s")
    pltpu.sync_copy(zeros, agg_sh.at[pl.ds(s * STRIPE, STRIPE)])
    pltpu.sync_copy(src2d.at[pl.ds(s * ROWS_S, ROWS_S)], sidx)
    pltpu.sync_copy(dst2d.at[pl.ds(s * ROWS_S, ROWS_S)], didx)
    plsc.subcore_barrier()

    def edge_loop(tbl):
        pltpu.async_copy(tbl.at[sidx.at[0]], buf0, sem0)

        @pl.loop(0, ROWS_S, step=2)
        def _(k):
            pltpu.make_async_copy(tbl.at[sidx.at[k]], buf0, sem0).wait()
            pltpu.async_copy(tbl.at[sidx.at[k + 1]], buf1, sem1)
            pltpu.sync_copy(buf0, agg_sh.at[didx.at[k]], add=True)
            pltpu.make_async_copy(tbl.at[sidx.at[k + 1]], buf1, sem1).wait()

            @pl.when(k + 2 < ROWS_S)
            def _():
                pltpu.async_copy(tbl.at[sidx.at[k + 2]], buf0, sem0)

            pltpu.sync_copy(buf1, agg_sh.at[didx.at[k + 1]], add=True)

    @pl.when(c == 0)
    def _():
        edge_loop(hs_a)

    @pl.when(c == 1)
    def _():
        edge_loop(hs_b)

    plsc.subcore_barrier()
    pltpu.sync_copy(agg_sh.at[pl.ds(s * STRIPE, STRIPE)],
                    part.at[c, pl.ds(s * STRIPE, STRIPE)])


def _tc_prep_body(x_ref, we_ref, be_ref, degp_ref, hsa_ref, hsb_ref,
                  dinv_ref):
    x = x_ref[...]
    h = jnp.maximum(
        jnp.dot(x, we_ref[...], preferred_element_type=jnp.float32)
        + be_ref[...], 0.0)
    deg = degp_ref[0, :, 0] + degp_ref[1, :, 0] + 1.0
    dinv = lax.rsqrt(jnp.maximum(deg, 1.0))
    dinv_col = dinv[:, None]
    dinv_ref[...] = dinv_col
    hs = h * dinv_col[0:NN, :]
    hsa_ref[0:NN, :] = hs[:, 0:DH]
    hsa_ref[NN:NP, :] = jnp.zeros((NP - NN, DH), jnp.float32)
    hsb_ref[0:NN, :] = hs[:, DH:D]
    hsb_ref[NN:NP, :] = jnp.zeros((NP - NN, DH), jnp.float32)


def _tc_conv_body(part_ref, hsa_ref, hsb_ref, dinv_ref, w_ref, b_ref, g_ref,
                  be_ref, outa_ref, outb_ref):
    dinv = dinv_ref[...]
    ssum = jnp.concatenate(
        [part_ref[0] + hsa_ref[...], part_ref[1] + hsb_ref[...]], axis=1)
    agg = ssum[0:NN] * dinv[0:NN]
    z = jnp.dot(agg, w_ref[...], preferred_element_type=jnp.float32) + b_ref[...]
    mu = jnp.mean(z, axis=0, keepdims=True)
    zc = z - mu
    var = jnp.mean(zc * zc, axis=0, keepdims=True)
    h = jnp.maximum(zc * lax.rsqrt(var + 1e-5) * g_ref[...] + be_ref[...], 0.0)
    hs = h * dinv[0:NN]
    outa_ref[0:NN, :] = hs[:, 0:DH]
    outa_ref[NN:NP, :] = jnp.zeros((NP - NN, DH), jnp.float32)
    outb_ref[0:NN, :] = hs[:, DH:D]
    outb_ref[NN:NP, :] = jnp.zeros((NP - NN, DH), jnp.float32)


def _tc_final_body(part_ref, hsa_ref, hsb_ref, dinv_ref, w3_ref, b3_ref,
                   batch_ref, wd1_ref, bd1_ref, wd2_ref, bd2_ref, out_ref):
    dinv = dinv_ref[...]
    ssum = jnp.concatenate(
        [part_ref[0] + hsa_ref[...], part_ref[1] + hsb_ref[...]], axis=1)
    agg = ssum[0:NN] * dinv[0:NN]
    z = jnp.dot(agg, w3_ref[...], preferred_element_type=jnp.float32) + b3_ref[...]
    b = batch_ref[...]
    gi = lax.broadcasted_iota(jnp.int32, (NG, NN), 0)
    oh = jnp.where(b == gi, 1.0, 0.0)
    ssel = jnp.dot(oh, z, preferred_element_type=jnp.float32)
    cnt = jnp.sum(oh, axis=1, keepdims=True)
    pooled = ssel / jnp.maximum(cnt, 1.0)
    t = jnp.maximum(
        jnp.dot(pooled, wd1_ref[...], preferred_element_type=jnp.float32)
        + bd1_ref[...], 0.0)
    out_ref[...] = (jnp.dot(t, wd2_ref[...], preferred_element_type=jnp.float32)
                    + bd2_ref[...])


def _f32(shape):
    return jax.ShapeDtypeStruct(shape, jnp.float32)


@jax.jit
def _impl(x, edge_index, batch, W_enc, b_enc, W1, b1, g1, be1, W2, b2, g2,
          be2, W3, b3, Wd1, bd1, Wd2, bd2):
    src = edge_index[0]
    dst = edge_index[1]
    e = src.shape[0]
    pad = EP - e
    src2 = jnp.concatenate(
        [src, jnp.full((pad,), NN, jnp.int32)]).reshape(EP // CHUNK, CHUNK)
    dst2 = jnp.concatenate(
        [dst, jnp.full((pad,), NN, jnp.int32)]).reshape(EP // CHUNK, CHUNK)
    zeros = jnp.zeros((STRIPE, DH), jnp.float32)
    zeros16 = jnp.zeros((STRIPE, 16), jnp.float32)
    ones16 = jnp.ones((CHUNK, 16), jnp.float32)

    degp = _sc_degree(dst2, ones16, zeros16)
    hsa, hsb, dinv = pl.pallas_call(
        _tc_prep_body,
        out_shape=[_f32((NP, DH)), _f32((NP, DH)), _f32((NP, 1))],
    )(x, W_enc, b_enc.reshape(1, D), degp)

    for (W, bb, g, be) in ((W1, b1, g1, be1), (W2, b2, g2, be2)):
        part = _sc_agg(src2, dst2, hsa, hsb, zeros)
        hsa, hsb = pl.pallas_call(
            _tc_conv_body,
            out_shape=[_f32((NP, DH)), _f32((NP, DH))],
        )(part, hsa, hsb, dinv, W, bb.reshape(1, D), g.reshape(1, D),
          be.reshape(1, D))

    part = _sc_agg(src2, dst2, hsa, hsb, zeros)
    out = pl.pallas_call(
        _tc_final_body,
        out_shape=_f32((NG, NCLS)),
    )(part, hsa, hsb, dinv, W3, b3.reshape(1, D), batch.reshape(1, NN), Wd1,
      bd1.reshape(1, D), Wd2, bd2.reshape(1, NCLS))
    return out


def kernel(x, edge_index, batch, W_enc, b_enc, W1, b1, g1, be1, W2, b2, g2,
           be2, W3, b3, Wd1, bd1, Wd2, bd2):
    return _impl(x, edge_index, batch, W_enc, b_enc, W1, b1, g1, be1, W2, b2,
                 g2, be2, W3, b3, Wd1, bd1, Wd2, bd2)


# speedup vs baseline: 11.3654x; 1.1316x over previous
import functools

import jax
import jax.numpy as jnp
from jax import lax
from jax.experimental import pallas as pl
from jax.experimental.pallas import tpu as pltpu
from jax.experimental.pallas import tpu_sc as plsc

NN = 10000
D = 128
DH = D // 2
NG = 16
NCLS = 10
NP = 10240
NC, NS = 2, 16
NW = NC * NS
CHUNK = 128
EP = 327680
ROWS_S = EP // NS // CHUNK
ROWS_W = EP // NW // CHUNK
STRIPE = NP // NS

_mesh = plsc.VectorSubcoreMesh(core_axis_name="c", subcore_axis_name="s")


@functools.partial(
    pl.kernel,
    out_type=jax.ShapeDtypeStruct((NC, NP, 16), jnp.float32),
    mesh=_mesh,
    compiler_params=pltpu.CompilerParams(use_tc_tiling_on_sc=False),
    scratch_types=[
        pltpu.VMEM((ROWS_W, CHUNK), jnp.int32),
        pltpu.VMEM((CHUNK, 16), jnp.float32),
        pltpu.VMEM_SHARED((NP, 16), jnp.float32),
    ],
)
def _sc_degree(dst2d, ones, zeros16, degp, didx, ones_v, deg_sh):
    c = lax.axis_index("c")
    s = lax.axis_index("s")
    wid = c * NS + s
    pltpu.sync_copy(zeros16, deg_sh.at[pl.ds(s * STRIPE, STRIPE)])
    pltpu.sync_copy(ones, ones_v)
    pltpu.sync_copy(dst2d.at[pl.ds(wid * ROWS_W, ROWS_W)], didx)
    plsc.subcore_barrier()

    @pl.loop(0, ROWS_W)
    def _(k):
        pltpu.sync_copy(ones_v, deg_sh.at[didx.at[k]], add=True)

    plsc.subcore_barrier()
    pltpu.sync_copy(deg_sh.at[pl.ds(s * STRIPE, STRIPE)],
                    degp.at[c, pl.ds(s * STRIPE, STRIPE)])


NBUF = 4


@functools.partial(
    pl.kernel,
    out_type=jax.ShapeDtypeStruct((NC, NP, DH), jnp.float32),
    mesh=_mesh,
    compiler_params=pltpu.CompilerParams(use_tc_tiling_on_sc=False),
    scratch_types=[
        pltpu.VMEM((ROWS_S, CHUNK), jnp.int32),
        pltpu.VMEM((ROWS_S, CHUNK), jnp.int32),
        [pltpu.VMEM((CHUNK, DH), jnp.float32) for _ in range(NBUF)],
        pltpu.VMEM_SHARED((NP, DH), jnp.float32),
        [pltpu.SemaphoreType.DMA for _ in range(NBUF)],
        [pltpu.SemaphoreType.DMA for _ in range(NBUF)],
    ],
)
def _sc_agg(src2d, dst2d, hs_a, hs_b, zeros, part, sidx, didx, bufs, agg_sh,
            sems_g, sems_s):
    c = lax.axis_index("c")
    s = lax.axis_index("s")
    pltpu.sync_copy(zeros, agg_sh.at[pl.ds(s * STRIPE, STRIPE)])
    pltpu.sync_copy(src2d.at[pl.ds(s * ROWS_S, ROWS_S)], sidx)
    pltpu.sync_copy(dst2d.at[pl.ds(s * ROWS_S, ROWS_S)], didx)
    plsc.subcore_barrier()

    def edge_loop(tbl):
        for j in range(NBUF):
            pltpu.async_copy(tbl.at[sidx.at[j]], bufs[j], sems_g[j])

        @pl.loop(0, ROWS_S, step=NBUF)
        def _(k):
            descs = []
            for j in range(NBUF):
                pltpu.make_async_copy(
                    tbl.at[sidx.at[k + j]], bufs[j], sems_g[j]).wait()
                descs.append(pltpu.async_copy(
                    bufs[j], agg_sh.at[didx.at[k + j]], sems_s[j], add=True))
            for j in range(NBUF):
                descs[j].wait()

                @pl.when(k + j + NBUF < ROWS_S)
                def _():
                    pltpu.async_copy(
                        tbl.at[sidx.at[k + j + NBUF]], bufs[j], sems_g[j])

    @pl.when(c == 0)
    def _():
        edge_loop(hs_a)

    @pl.when(c == 1)
    def _():
        edge_loop(hs_b)

    plsc.subcore_barrier()
    pltpu.sync_copy(agg_sh.at[pl.ds(s * STRIPE, STRIPE)],
                    part.at[c, pl.ds(s * STRIPE, STRIPE)])


def _tc_prep_body(x_ref, we_ref, be_ref, degp_ref, hsa_ref, hsb_ref,
                  dinv_ref):
    x = x_ref[...]
    h = jnp.maximum(
        jnp.dot(x, we_ref[...], preferred_element_type=jnp.float32)
        + be_ref[...], 0.0)
    deg = degp_ref[0, :, 0] + degp_ref[1, :, 0] + 1.0
    dinv = lax.rsqrt(jnp.maximum(deg, 1.0))
    dinv_col = dinv[:, None]
    dinv_ref[...] = dinv_col
    hs = h * dinv_col[0:NN, :]
    hsa_ref[0:NN, :] = hs[:, 0:DH]
    hsa_ref[NN:NP, :] = jnp.zeros((NP - NN, DH), jnp.float32)
    hsb_ref[0:NN, :] = hs[:, DH:D]
    hsb_ref[NN:NP, :] = jnp.zeros((NP - NN, DH), jnp.float32)


def _tc_conv_body(part_ref, hsa_ref, hsb_ref, dinv_ref, w_ref, b_ref, g_ref,
                  be_ref, outa_ref, outb_ref):
    dinv = dinv_ref[...]
    ssum = jnp.concatenate(
        [part_ref[0] + hsa_ref[...], part_ref[1] + hsb_ref[...]], axis=1)
    agg = ssum[0:NN] * dinv[0:NN]
    z = jnp.dot(agg, w_ref[...], preferred_element_type=jnp.float32) + b_ref[...]
    mu = jnp.mean(z, axis=0, keepdims=True)
    zc = z - mu
    var = jnp.mean(zc * zc, axis=0, keepdims=True)
    h = jnp.maximum(zc * lax.rsqrt(var + 1e-5) * g_ref[...] + be_ref[...], 0.0)
    hs = h * dinv[0:NN]
    outa_ref[0:NN, :] = hs[:, 0:DH]
    outa_ref[NN:NP, :] = jnp.zeros((NP - NN, DH), jnp.float32)
    outb_ref[0:NN, :] = hs[:, DH:D]
    outb_ref[NN:NP, :] = jnp.zeros((NP - NN, DH), jnp.float32)


def _tc_final_body(part_ref, hsa_ref, hsb_ref, dinv_ref, w3_ref, b3_ref,
                   batch_ref, wd1_ref, bd1_ref, wd2_ref, bd2_ref, out_ref):
    dinv = dinv_ref[...]
    ssum = jnp.concatenate(
        [part_ref[0] + hsa_ref[...], part_ref[1] + hsb_ref[...]], axis=1)
    agg = ssum[0:NN] * dinv[0:NN]
    z = jnp.dot(agg, w3_ref[...], preferred_element_type=jnp.float32) + b3_ref[...]
    b = batch_ref[...]
    gi = lax.broadcasted_iota(jnp.int32, (NG, NN), 0)
    oh = jnp.where(b == gi, 1.0, 0.0)
    ssel = jnp.dot(oh, z, preferred_element_type=jnp.float32)
    cnt = jnp.sum(oh, axis=1, keepdims=True)
    pooled = ssel / jnp.maximum(cnt, 1.0)
    t = jnp.maximum(
        jnp.dot(pooled, wd1_ref[...], preferred_element_type=jnp.float32)
        + bd1_ref[...], 0.0)
    out_ref[...] = (jnp.dot(t, wd2_ref[...], preferred_element_type=jnp.float32)
                    + bd2_ref[...])


def _f32(shape):
    return jax.ShapeDtypeStruct(shape, jnp.float32)


@jax.jit
def _impl(x, edge_index, batch, W_enc, b_enc, W1, b1, g1, be1, W2, b2, g2,
          be2, W3, b3, Wd1, bd1, Wd2, bd2):
    src = edge_index[0]
    dst = edge_index[1]
    e = src.shape[0]
    pad = EP - e
    src2 = jnp.concatenate(
        [src, jnp.full((pad,), NN, jnp.int32)]).reshape(EP // CHUNK, CHUNK)
    dst2 = jnp.concatenate(
        [dst, jnp.full((pad,), NN, jnp.int32)]).reshape(EP // CHUNK, CHUNK)
    zeros = jnp.zeros((STRIPE, DH), jnp.float32)
    zeros16 = jnp.zeros((STRIPE, 16), jnp.float32)
    ones16 = jnp.ones((CHUNK, 16), jnp.float32)

    degp = _sc_degree(dst2, ones16, zeros16)
    hsa, hsb, dinv = pl.pallas_call(
        _tc_prep_body,
        out_shape=[_f32((NP, DH)), _f32((NP, DH)), _f32((NP, 1))],
    )(x, W_enc, b_enc.reshape(1, D), degp)

    for (W, bb, g, be) in ((W1, b1, g1, be1), (W2, b2, g2, be2)):
        part = _sc_agg(src2, dst2, hsa, hsb, zeros)
        hsa, hsb = pl.pallas_call(
            _tc_conv_body,
            out_shape=[_f32((NP, DH)), _f32((NP, DH))],
        )(part, hsa, hsb, dinv, W, bb.reshape(1, D), g.reshape(1, D),
          be.reshape(1, D))

    part = _sc_agg(src2, dst2, hsa, hsb, zeros)
    out = pl.pallas_call(
        _tc_final_body,
        out_shape=_f32((NG, NCLS)),
    )(part, hsa, hsb, dinv, W3, b3.reshape(1, D), batch.reshape(1, NN), Wd1,
      bd1.reshape(1, D), Wd2, bd2.reshape(1, NCLS))
    return out


def kernel(x, edge_index, batch, W_enc, b_enc, W1, b1, g1, be1, W2, b2, g2,
           be2, W3, b3, Wd1, bd1, Wd2, bd2):
    return _impl(x, edge_index, batch, W_enc, b_enc, W1, b1, g1, be1, W2, b2,
                 g2, be2, W3, b3, Wd1, bd1, Wd2, bd2)


# speedup vs baseline: 11.4074x; 1.0037x over previous
import functools

import jax
import jax.numpy as jnp
from jax import lax
from jax.experimental import pallas as pl
from jax.experimental.pallas import tpu as pltpu
from jax.experimental.pallas import tpu_sc as plsc

NN = 10000
D = 128
DH = D // 2
NG = 16
NCLS = 10
NP = 10240
NC, NS = 2, 16
NW = NC * NS
CHUNK = 128
EP = 327680
ROWS_S = EP // NS // CHUNK
ROWS_W = EP // NW // CHUNK
STRIPE = NP // NS

_mesh = plsc.VectorSubcoreMesh(core_axis_name="c", subcore_axis_name="s")


@functools.partial(
    pl.kernel,
    out_type=jax.ShapeDtypeStruct((NC, NP, 16), jnp.float32),
    mesh=_mesh,
    compiler_params=pltpu.CompilerParams(use_tc_tiling_on_sc=False),
    scratch_types=[
        pltpu.VMEM((ROWS_W, CHUNK), jnp.int32),
        pltpu.VMEM((CHUNK, 16), jnp.float32),
        pltpu.VMEM_SHARED((NP, 16), jnp.float32),
    ],
)
def _sc_degree(dst2d, ones, zeros16, degp, didx, ones_v, deg_sh):
    c = lax.axis_index("c")
    s = lax.axis_index("s")
    wid = c * NS + s
    pltpu.sync_copy(zeros16, deg_sh.at[pl.ds(s * STRIPE, STRIPE)])
    pltpu.sync_copy(ones, ones_v)
    pltpu.sync_copy(dst2d.at[pl.ds(wid * ROWS_W, ROWS_W)], didx)
    plsc.subcore_barrier()

    @pl.loop(0, ROWS_W)
    def _(k):
        pltpu.sync_copy(ones_v, deg_sh.at[didx.at[k]], add=True)

    plsc.subcore_barrier()
    pltpu.sync_copy(deg_sh.at[pl.ds(s * STRIPE, STRIPE)],
                    degp.at[c, pl.ds(s * STRIPE, STRIPE)])


NBUF = 5


@functools.partial(
    pl.kernel,
    out_type=jax.ShapeDtypeStruct((NC, NP, DH), jnp.float32),
    mesh=_mesh,
    compiler_params=pltpu.CompilerParams(use_tc_tiling_on_sc=False),
    scratch_types=[
        pltpu.VMEM((ROWS_S, CHUNK), jnp.int32),
        pltpu.VMEM((ROWS_S, CHUNK), jnp.int32),
        [pltpu.VMEM((CHUNK, DH), jnp.float32) for _ in range(NBUF)],
        pltpu.VMEM_SHARED((NP, DH), jnp.float32),
        [pltpu.SemaphoreType.DMA for _ in range(NBUF)],
        [pltpu.SemaphoreType.DMA for _ in range(NBUF)],
    ],
)
def _sc_agg(src2d, dst2d, hs_a, hs_b, zeros, part, sidx, didx, bufs, agg_sh,
            sems_g, sems_s):
    c = lax.axis_index("c")
    s = lax.axis_index("s")
    pltpu.sync_copy(zeros, agg_sh.at[pl.ds(s * STRIPE, STRIPE)])
    pltpu.sync_copy(src2d.at[pl.ds(s * ROWS_S, ROWS_S)], sidx)
    pltpu.sync_copy(dst2d.at[pl.ds(s * ROWS_S, ROWS_S)], didx)
    plsc.subcore_barrier()

    def edge_loop(tbl):
        for j in range(NBUF):
            pltpu.async_copy(tbl.at[sidx.at[j]], bufs[j], sems_g[j])

        @pl.loop(0, ROWS_S, step=NBUF)
        def _(k):
            descs = []
            for j in range(NBUF):
                pltpu.make_async_copy(
                    tbl.at[sidx.at[k + j]], bufs[j], sems_g[j]).wait()
                descs.append(pltpu.async_copy(
                    bufs[j], agg_sh.at[didx.at[k + j]], sems_s[j], add=True))
            for j in range(NBUF):
                descs[j].wait()

                @pl.when(k + j + NBUF < ROWS_S)
                def _():
                    pltpu.async_copy(
                        tbl.at[sidx.at[k + j + NBUF]], bufs[j], sems_g[j])

    @pl.when(c == 0)
    def _():
        edge_loop(hs_a)

    @pl.when(c == 1)
    def _():
        edge_loop(hs_b)

    plsc.subcore_barrier()
    pltpu.sync_copy(agg_sh.at[pl.ds(s * STRIPE, STRIPE)],
                    part.at[c, pl.ds(s * STRIPE, STRIPE)])


def _tc_prep_body(x_ref, we_ref, be_ref, degp_ref, hsa_ref, hsb_ref,
                  dinv_ref):
    x = x_ref[...]
    h = jnp.maximum(
        jnp.dot(x, we_ref[...], preferred_element_type=jnp.float32)
        + be_ref[...], 0.0)
    deg = degp_ref[0, :, 0] + degp_ref[1, :, 0] + 1.0
    dinv = lax.rsqrt(jnp.maximum(deg, 1.0))
    dinv_col = dinv[:, None]
    dinv_ref[...] = dinv_col
    hs = h * dinv_col[0:NN, :]
    hsa_ref[0:NN, :] = hs[:, 0:DH]
    hsa_ref[NN:NP, :] = jnp.zeros((NP - NN, DH), jnp.float32)
    hsb_ref[0:NN, :] = hs[:, DH:D]
    hsb_ref[NN:NP, :] = jnp.zeros((NP - NN, DH), jnp.float32)


def _tc_conv_body(part_ref, hsa_ref, hsb_ref, dinv_ref, w_ref, b_ref, g_ref,
                  be_ref, outa_ref, outb_ref):
    dinv = dinv_ref[...]
    ssum = jnp.concatenate(
        [part_ref[0] + hsa_ref[...], part_ref[1] + hsb_ref[...]], axis=1)
    agg = ssum[0:NN] * dinv[0:NN]
    z = jnp.dot(agg, w_ref[...], preferred_element_type=jnp.float32) + b_ref[...]
    mu = jnp.mean(z, axis=0, keepdims=True)
    zc = z - mu
    var = jnp.mean(zc * zc, axis=0, keepdims=True)
    h = jnp.maximum(zc * lax.rsqrt(var + 1e-5) * g_ref[...] + be_ref[...], 0.0)
    hs = h * dinv[0:NN]
    outa_ref[0:NN, :] = hs[:, 0:DH]
    outa_ref[NN:NP, :] = jnp.zeros((NP - NN, DH), jnp.float32)
    outb_ref[0:NN, :] = hs[:, DH:D]
    outb_ref[NN:NP, :] = jnp.zeros((NP - NN, DH), jnp.float32)


def _tc_final_body(part_ref, hsa_ref, hsb_ref, dinv_ref, w3_ref, b3_ref,
                   batch_ref, wd1_ref, bd1_ref, wd2_ref, bd2_ref, out_ref):
    dinv = dinv_ref[...]
    ssum = jnp.concatenate(
        [part_ref[0] + hsa_ref[...], part_ref[1] + hsb_ref[...]], axis=1)
    agg = ssum[0:NN] * dinv[0:NN]
    z = jnp.dot(agg, w3_ref[...], preferred_element_type=jnp.float32) + b3_ref[...]
    b = batch_ref[...]
    gi = lax.broadcasted_iota(jnp.int32, (NG, NN), 0)
    oh = jnp.where(b == gi, 1.0, 0.0)
    ssel = jnp.dot(oh, z, preferred_element_type=jnp.float32)
    cnt = jnp.sum(oh, axis=1, keepdims=True)
    pooled = ssel / jnp.maximum(cnt, 1.0)
    t = jnp.maximum(
        jnp.dot(pooled, wd1_ref[...], preferred_element_type=jnp.float32)
        + bd1_ref[...], 0.0)
    out_ref[...] = (jnp.dot(t, wd2_ref[...], preferred_element_type=jnp.float32)
                    + bd2_ref[...])


def _f32(shape):
    return jax.ShapeDtypeStruct(shape, jnp.float32)


@jax.jit
def _impl(x, edge_index, batch, W_enc, b_enc, W1, b1, g1, be1, W2, b2, g2,
          be2, W3, b3, Wd1, bd1, Wd2, bd2):
    src = edge_index[0]
    dst = edge_index[1]
    e = src.shape[0]
    pad = EP - e
    src2 = jnp.concatenate(
        [src, jnp.full((pad,), NN, jnp.int32)]).reshape(EP // CHUNK, CHUNK)
    dst2 = jnp.concatenate(
        [dst, jnp.full((pad,), NN, jnp.int32)]).reshape(EP // CHUNK, CHUNK)
    zeros = jnp.zeros((STRIPE, DH), jnp.float32)
    zeros16 = jnp.zeros((STRIPE, 16), jnp.float32)
    ones16 = jnp.ones((CHUNK, 16), jnp.float32)

    degp = _sc_degree(dst2, ones16, zeros16)
    hsa, hsb, dinv = pl.pallas_call(
        _tc_prep_body,
        out_shape=[_f32((NP, DH)), _f32((NP, DH)), _f32((NP, 1))],
    )(x, W_enc, b_enc.reshape(1, D), degp)

    for (W, bb, g, be) in ((W1, b1, g1, be1), (W2, b2, g2, be2)):
        part = _sc_agg(src2, dst2, hsa, hsb, zeros)
        hsa, hsb = pl.pallas_call(
            _tc_conv_body,
            out_shape=[_f32((NP, DH)), _f32((NP, DH))],
        )(part, hsa, hsb, dinv, W, bb.reshape(1, D), g.reshape(1, D),
          be.reshape(1, D))

    part = _sc_agg(src2, dst2, hsa, hsb, zeros)
    out = pl.pallas_call(
        _tc_final_body,
        out_shape=_f32((NG, NCLS)),
    )(part, hsa, hsb, dinv, W3, b3.reshape(1, D), batch.reshape(1, NN), Wd1,
      bd1.reshape(1, D), Wd2, bd2.reshape(1, NCLS))
    return out


def kernel(x, edge_index, batch, W_enc, b_enc, W1, b1, g1, be1, W2, b2, g2,
           be2, W3, b3, Wd1, bd1, Wd2, bd2):
    return _impl(x, edge_index, batch, W_enc, b_enc, W1, b1, g1, be1, W2, b2,
                 g2, be2, W3, b3, Wd1, bd1, Wd2, bd2)
